# baseline (speedup 1.0000x reference)
import jax
import jax.numpy as jnp
from jax import lax
from jax.experimental import pallas as pl
from jax.experimental.pallas import tpu as pltpu
from jax.experimental.pallas import tpu_sc as plsc

_T, _E = 8192, 1024
_NC, _NS = 2, 16
_NW = _NC * _NS
_ROWS_PER_W = _T // _NW
_CR = 32
_NCHUNKS = _ROWS_PER_W // _CR


def _sc_copy(wpe_hbm, out_hbm, buf0, buf1, si0, si1, so0, so1):
    wid = lax.axis_index("s") * _NC + lax.axis_index("c")
    base = wid * _ROWS_PER_W
    bufs = (buf0, buf1)
    isems = (si0, si1)
    osems = (so0, so1)

    def src(k):
        return wpe_hbm.at[pl.ds(base + k * _CR, _CR)]

    def dst(k):
        return out_hbm.at[pl.ds(base + k * _CR, _CR)]

    in_copies = [None] * _NCHUNKS
    out_copies = [None] * _NCHUNKS
    in_copies[0] = pltpu.async_copy(src(0), bufs[0], isems[0])
    for k in range(_NCHUNKS):
        b = k % 2
        in_copies[k].wait()
        if k >= 1:
            out_copies[k - 1].wait()
        if k + 1 < _NCHUNKS:
            in_copies[k + 1] = pltpu.async_copy(
                src(k + 1), bufs[1 - b], isems[1 - b]
            )
        out_copies[k] = pltpu.async_copy(bufs[b], dst(k), osems[b])
    out_copies[_NCHUNKS - 1].wait()


_sc_lookup = pl.kernel(
    _sc_copy,
    out_type=jax.ShapeDtypeStruct((_T, _E), jnp.float32),
    mesh=plsc.VectorSubcoreMesh(core_axis_name="c", subcore_axis_name="s"),
    scratch_types=[
        pltpu.VMEM((_CR, _E), jnp.float32),
        pltpu.VMEM((_CR, _E), jnp.float32),
        pltpu.SemaphoreType.DMA,
        pltpu.SemaphoreType.DMA,
        pltpu.SemaphoreType.DMA,
        pltpu.SemaphoreType.DMA,
    ],
)


def kernel(idx, wpe):
    del idx
    return _sc_lookup(wpe)[None, :, :]


# speedup vs baseline: 1.0304x; 1.0304x over previous
import jax
import jax.numpy as jnp
from jax import lax
from jax.experimental import pallas as pl
from jax.experimental.pallas import tpu as pltpu
from jax.experimental.pallas import tpu_sc as plsc

_T, _E = 8192, 1024
_NC, _NS = 2, 16
_NW = _NC * _NS
_ROWS_PER_W = _T // _NW
_CR = 64
_NCHUNKS = _ROWS_PER_W // _CR


def _sc_copy(wpe_hbm, out_hbm, buf0, si0, so0):
    wid = lax.axis_index("s") * _NC + lax.axis_index("c")
    base = wid * _ROWS_PER_W

    def src(k):
        return wpe_hbm.at[pl.ds(base + k * _CR, _CR)]

    def dst(k):
        return out_hbm.at[pl.ds(base + k * _CR, _CR)]

    for k in range(_NCHUNKS):
        pltpu.async_copy(src(k), buf0, si0).wait()
        pltpu.async_copy(buf0, dst(k), so0).wait()


_sc_lookup = pl.kernel(
    _sc_copy,
    out_type=jax.ShapeDtypeStruct((_T, _E), jnp.float32),
    mesh=plsc.VectorSubcoreMesh(core_axis_name="c", subcore_axis_name="s"),
    scratch_types=[
        pltpu.VMEM((_CR, _E), jnp.float32),
        pltpu.SemaphoreType.DMA,
        pltpu.SemaphoreType.DMA,
    ],
)


def kernel(idx, wpe):
    del idx
    return _sc_lookup(wpe)[None, :, :]


# speedup vs baseline: 1.0381x; 1.0074x over previous
import jax
import jax.numpy as jnp
from jax import lax
from jax.experimental import pallas as pl
from jax.experimental.pallas import tpu as pltpu
from jax.experimental.pallas import tpu_sc as plsc

_T, _E = 8192, 1024
_NC, _NS = 2, 16
_NW = _NC * _NS
_ROWS_PER_W = _T // _NW
_CR = 64
_NCHUNKS = _ROWS_PER_W // _CR


def _sc_copy(wpe_hbm, out_hbm, buf0, si0, so0):
    wid = lax.axis_index("s") * _NC + lax.axis_index("c")
    base = wid * _ROWS_PER_W

    def src(k):
        return wpe_hbm.at[pl.ds(base + k * _CR, _CR)]

    def dst(k):
        return out_hbm.at[0, pl.ds(base + k * _CR, _CR)]

    for k in range(_NCHUNKS):
        pltpu.async_copy(src(k), buf0, si0).wait()
        pltpu.async_copy(buf0, dst(k), so0).wait()


_sc_lookup = pl.kernel(
    _sc_copy,
    out_type=jax.ShapeDtypeStruct((1, _T, _E), jnp.float32),
    mesh=plsc.VectorSubcoreMesh(core_axis_name="c", subcore_axis_name="s"),
    scratch_types=[
        pltpu.VMEM((_CR, _E), jnp.float32),
        pltpu.SemaphoreType.DMA,
        pltpu.SemaphoreType.DMA,
    ],
)


def kernel(idx, wpe):
    del idx
    return _sc_lookup(wpe)


# speedup vs baseline: 1.8214x; 1.7545x over previous
import jax
import jax.numpy as jnp
from jax.experimental import pallas as pl


def _copy_body(w_ref, o_ref):
    o_ref[...] = w_ref[...]


def kernel(idx, wpe):
    del idx
    T, E = wpe.shape
    BR = 512
    out = pl.pallas_call(
        _copy_body,
        grid=(T // BR,),
        in_specs=[pl.BlockSpec((BR, E), lambda i: (i, 0))],
        out_specs=pl.BlockSpec((BR, E), lambda i: (i, 0)),
        out_shape=jax.ShapeDtypeStruct((T, E), wpe.dtype),
    )(wpe)
    return out[None, :, :]
